# baseline (device time: 20688 ns/iter reference)
import jax
import jax.numpy as jnp
from jax import lax
from jax.experimental import pallas as pl
from jax.experimental.pallas import tpu as pltpu

N_DEV = 4
E_LOCAL = 4
R = 4
WIRE = jnp.float8_e4m3fn


def kernel(x, router_W, route_idx, expert_W, shared_W):
    n_tok, d_model = x.shape
    n_exp = router_W.shape[1]
    d_ff = expert_W.shape[2]
    rows = n_tok // R

    def body(x_ref, rw_ref, ridx_ref, ew_ref, sw_ref, out_ref,
             comm, send_sems, recv_sems):
        my_pos = lax.axis_index("i")
        px = 3 - my_pos
        py = my_pos ^ 1

        barrier_sem = pltpu.get_barrier_semaphore()
        for nbr in [px, py]:
            pl.semaphore_signal(
                barrier_sem, inc=1,
                device_id=(nbr,), device_id_type=pl.DeviceIdType.MESH,
            )
        pl.semaphore_wait(barrier_sem, 2)

        xv = x_ref[:, :]
        scores = jnp.dot(xv, rw_ref[:, :], preferred_element_type=jnp.float32)
        s_max = jnp.max(scores, axis=1, keepdims=True)
        p = jnp.exp(scores - s_max)
        probs = p / jnp.sum(p, axis=1, keepdims=True)
        idx = ridx_ref[:, :]
        eids = lax.broadcasted_iota(jnp.int32, (n_tok, n_exp), 1)
        w = jnp.sum(jnp.where(eids == idx, probs, 0.0), axis=1, keepdims=True)

        ew16 = [ew_ref[le].astype(jnp.bfloat16) for le in range(E_LOCAL)]
        xv16 = xv.astype(jnp.bfloat16)

        def step_partner(b, step):
            first, second = (px, py) if b < R // 2 else (py, px)
            return first if step == 0 else second

        def exchange(slot_src, slot_dst, b, step):
            return pltpu.make_async_remote_copy(
                src_ref=comm.at[slot_src, b],
                dst_ref=comm.at[slot_dst, b],
                send_sem=send_sems.at[step * R + b],
                recv_sem=recv_sems.at[step * R + b],
                device_id=(step_partner(b, step),),
                device_id_type=pl.DeviceIdType.MESH,
            )

        r1 = []
        for b in range(R):
            lo = b * rows
            xb = xv[lo:lo + rows, :]
            idx_b = idx[lo:lo + rows, :]
            w_b = w[lo:lo + rows, :]
            pblk = jnp.zeros((rows, d_ff), jnp.float32)
            for le in range(E_LOCAL):
                gate = jnp.where(idx_b == my_pos * E_LOCAL + le, w_b, 0.0)
                pblk = pblk + jnp.dot(
                    (xb * gate).astype(jnp.bfloat16), ew16[le],
                    preferred_element_type=jnp.float32,
                )
            comm[0, b] = pblk.astype(WIRE)
            r = exchange(0, 1, b, 0)
            r.start()
            r1.append(r)

        shared = jnp.dot(xv16, sw_ref[:, :].astype(jnp.bfloat16),
                         preferred_element_type=jnp.float32)

        r2 = []
        for b in range(R):
            r1[b].wait_recv()
            comm[2, b] = (
                comm[0, b, :, :].astype(jnp.float32)
                + comm[1, b, :, :].astype(jnp.float32)
            ).astype(WIRE)
            r = exchange(2, 3, b, 1)
            r.start()
            r2.append(r)

        for b in range(R):
            r2[b].wait_recv()
            lo = b * rows
            out_ref[lo:lo + rows, :] = (
                shared[lo:lo + rows, :]
                + comm[2, b, :, :].astype(jnp.float32)
                + comm[3, b, :, :].astype(jnp.float32)
            )

        for r in r1 + r2:
            r.wait_send()

    return pl.pallas_call(
        body,
        out_shape=jax.ShapeDtypeStruct((n_tok, d_ff), jnp.float32),
        in_specs=[pl.BlockSpec(memory_space=pltpu.VMEM)] * 5,
        out_specs=pl.BlockSpec(memory_space=pltpu.VMEM),
        scratch_shapes=[
            pltpu.VMEM((4, R, rows, d_ff), WIRE),
            pltpu.SemaphoreType.DMA((2 * R,)),
            pltpu.SemaphoreType.DMA((2 * R,)),
        ],
        compiler_params=pltpu.CompilerParams(collective_id=0),
    )(x, router_W, route_idx, expert_W, shared_W)
